# baseline (speedup 1.0000x reference)
import functools

import jax
import jax.numpy as jnp
from jax import lax
from jax.experimental import pallas as pl
from jax.experimental.pallas import tpu as pltpu
from jax.experimental.pallas import tpu_sc as plsc

N_NODES = 100000
BATCH = 8192
SRC_DIM = 128
DST_DIM = 128

_SC_INFO = plsc.get_sparse_core_info()
_NC = _SC_INFO.num_cores
_NS = _SC_INFO.num_subcores
_NW = _NC * _NS
_BPW = BATCH // _NW


def _make_sc_gather2():
    mesh = plsc.VectorSubcoreMesh(core_axis_name="c", subcore_axis_name="s")

    @functools.partial(
        pl.kernel,
        mesh=mesh,
        out_type=[
            jax.ShapeDtypeStruct((BATCH, SRC_DIM), jnp.float32),
            jax.ShapeDtypeStruct((BATCH, SRC_DIM), jnp.float32),
        ],
        scratch_types=[
            pltpu.VMEM((_BPW,), jnp.int32),
            pltpu.VMEM((_BPW,), jnp.int32),
            pltpu.VMEM((_BPW, SRC_DIM), jnp.float32),
            pltpu.VMEM((_BPW, SRC_DIM), jnp.float32),
            pltpu.SemaphoreType.DMA,
            pltpu.SemaphoreType.DMA,
            pltpu.SemaphoreType.DMA,
            pltpu.SemaphoreType.DMA,
            pltpu.SemaphoreType.DMA,
            pltpu.SemaphoreType.DMA,
        ],
    )
    def gather2(table_hbm, src_idx_hbm, dst_idx_hbm, src_out, dst_out,
                sidx_v, didx_v, srows_v, drows_v,
                sem_i1, sem_i2, sem_s, sem_d, sem_ws, sem_wd):
        wid = lax.axis_index("s") * _NC + lax.axis_index("c")
        base = wid * _BPW
        ci = pltpu.async_copy(src_idx_hbm.at[pl.ds(base, _BPW)], sidx_v, sem_i1)
        cj = pltpu.async_copy(dst_idx_hbm.at[pl.ds(base, _BPW)], didx_v, sem_i2)
        ci.wait()
        cp_s = pltpu.async_copy(table_hbm.at[sidx_v], srows_v, sem_s)
        cj.wait()
        cp_d = pltpu.async_copy(table_hbm.at[didx_v], drows_v, sem_d)
        cp_s.wait()
        ws = pltpu.async_copy(srows_v, src_out.at[pl.ds(base, _BPW)], sem_ws)
        cp_d.wait()
        wd = pltpu.async_copy(drows_v, dst_out.at[pl.ds(base, _BPW)], sem_wd)
        ws.wait()
        wd.wait()

    return gather2


_sc_gather2 = _make_sc_gather2()

_BLK = 256
_NBLK = BATCH // _BLK
_NBUF = 3
_QROWS = _BLK // 4
_QW = BATCH // 4


def _issue_block(dif_hbm, buf, sem, k, blk_idx):
    for q in range(4):
        pltpu.make_async_copy(
            dif_hbm.at[pl.ds(blk_idx * _BLK, _BLK), pl.ds(q * _QW, _QW)],
            buf.at[:, pl.ds(q * _QW, _QW)],
            sem.at[k, q],
        ).start()


def _wait_block(dif_hbm, buf, sem, k):
    for q in range(4):
        pltpu.make_async_copy(
            dif_hbm.at[pl.ds(0, _BLK), pl.ds(q * _QW, _QW)],
            buf.at[:, pl.ds(q * _QW, _QW)],
            sem.at[k, q],
        ).wait()


def _tc_manual_body(dif_hbm, src_hbm, dst_hbm, w_hbm, out_hbm,
                    buf0, buf1, buf2, src_v, dst_v, w_v, outacc,
                    sem, csem, osem):
    i = pl.program_id(0)
    bufs = (buf0, buf1, buf2)

    @pl.when(i == 0)
    def _prologue():
        pltpu.make_async_copy(src_hbm, src_v, csem.at[0]).start()
        pltpu.make_async_copy(dst_hbm, dst_v, csem.at[1]).start()
        pltpu.make_async_copy(w_hbm, w_v, csem.at[2]).start()
        for k in range(_NBUF):
            _issue_block(dif_hbm, bufs[k], sem, k, k)
        pltpu.make_async_copy(src_hbm, src_v, csem.at[0]).wait()
        pltpu.make_async_copy(dst_hbm, dst_v, csem.at[1]).wait()
        pltpu.make_async_copy(w_hbm, w_v, csem.at[2]).wait()

    slot = lax.rem(i, _NBUF)
    for k in range(_NBUF):
        @pl.when(slot == k)
        def _step(k=k):
            bufk = bufs[k]
            agg = None
            for q in range(4):
                pltpu.make_async_copy(
                    dif_hbm.at[pl.ds(0, _BLK), pl.ds(q * _QW, _QW)],
                    bufk.at[:, pl.ds(q * _QW, _QW)],
                    sem.at[k, q],
                ).wait()
                part = jnp.dot(bufk[:, q * _QW:(q + 1) * _QW],
                               src_v[q * _QW:(q + 1) * _QW, :],
                               preferred_element_type=jnp.float32)
                agg = part if agg is None else agg + part
            x = (jnp.dot(agg, w_v[:SRC_DIM, :],
                         preferred_element_type=jnp.float32)
                 + jnp.dot(dst_v[pl.ds(i * _BLK, _BLK), :], w_v[SRC_DIM:, :],
                           preferred_element_type=jnp.float32))
            outacc[pl.ds(i * _BLK, _BLK), :] = jnp.maximum(x, 0.0)
            pltpu.make_async_copy(
                outacc.at[pl.ds(i * _BLK, _BLK), :],
                out_hbm.at[pl.ds(i * _BLK, _BLK), :],
                osem.at[0],
            ).start()

            @pl.when(i < _NBLK - _NBUF)
            def _issue_next():
                _issue_block(dif_hbm, bufk, sem, k, i + _NBUF)

    @pl.when(i == _NBLK - 1)
    def _epilogue():
        pltpu.make_async_copy(outacc, out_hbm, osem.at[0]).wait()


def kernel(dstsrc_features, dstsrc2src, dstsrc2dst, dif_mat, w):
    src_f, dst_f = _sc_gather2(dstsrc_features, dstsrc2src, dstsrc2dst)
    out = pl.pallas_call(
        _tc_manual_body,
        grid=(_NBLK,),
        in_specs=[pl.BlockSpec(memory_space=pl.ANY)] * 4,
        out_specs=pl.BlockSpec(memory_space=pl.ANY),
        out_shape=jax.ShapeDtypeStruct((BATCH, DST_DIM), jnp.float32),
        scratch_shapes=[
            pltpu.VMEM((_BLK, BATCH), jnp.float32),
            pltpu.VMEM((_BLK, BATCH), jnp.float32),
            pltpu.VMEM((_BLK, BATCH), jnp.float32),
            pltpu.VMEM((BATCH, SRC_DIM), jnp.float32),
            pltpu.VMEM((BATCH, SRC_DIM), jnp.float32),
            pltpu.VMEM((2 * SRC_DIM, DST_DIM), jnp.float32),
            pltpu.VMEM((BATCH, DST_DIM), jnp.float32),
            pltpu.SemaphoreType.DMA((_NBUF, 4)),
            pltpu.SemaphoreType.DMA((3,)),
            pltpu.SemaphoreType.DMA((1,)),
        ],
    )(dif_mat, src_f, dst_f, w)
    return out


# speedup vs baseline: 1.1380x; 1.1380x over previous
import functools

import jax
import jax.numpy as jnp
from jax import lax
from jax.experimental import pallas as pl
from jax.experimental.pallas import tpu as pltpu
from jax.experimental.pallas import tpu_sc as plsc

N_NODES = 100000
BATCH = 8192
SRC_DIM = 128
DST_DIM = 128

_SC_INFO = plsc.get_sparse_core_info()
_NC = _SC_INFO.num_cores
_NS = _SC_INFO.num_subcores
_NW = _NC * _NS
_BPW = BATCH // _NW


def _make_sc_gather2():
    mesh = plsc.VectorSubcoreMesh(core_axis_name="c", subcore_axis_name="s")

    @functools.partial(
        pl.kernel,
        mesh=mesh,
        out_type=[
            jax.ShapeDtypeStruct((BATCH, SRC_DIM), jnp.float32),
            jax.ShapeDtypeStruct((BATCH, SRC_DIM), jnp.float32),
        ],
        scratch_types=[
            pltpu.VMEM((_BPW,), jnp.int32),
            pltpu.VMEM((_BPW,), jnp.int32),
            pltpu.VMEM((_BPW, SRC_DIM), jnp.float32),
            pltpu.VMEM((_BPW, SRC_DIM), jnp.float32),
            pltpu.SemaphoreType.DMA,
            pltpu.SemaphoreType.DMA,
            pltpu.SemaphoreType.DMA,
            pltpu.SemaphoreType.DMA,
            pltpu.SemaphoreType.DMA,
            pltpu.SemaphoreType.DMA,
        ],
    )
    def gather2(table_hbm, src_idx_hbm, dst_idx_hbm, src_out, dst_out,
                sidx_v, didx_v, srows_v, drows_v,
                sem_i1, sem_i2, sem_s, sem_d, sem_ws, sem_wd):
        wid = lax.axis_index("s") * _NC + lax.axis_index("c")
        base = wid * _BPW
        ci = pltpu.async_copy(src_idx_hbm.at[pl.ds(base, _BPW)], sidx_v, sem_i1)
        cj = pltpu.async_copy(dst_idx_hbm.at[pl.ds(base, _BPW)], didx_v, sem_i2)
        ci.wait()
        cp_s = pltpu.async_copy(table_hbm.at[sidx_v], srows_v, sem_s)
        cj.wait()
        cp_d = pltpu.async_copy(table_hbm.at[didx_v], drows_v, sem_d)
        cp_s.wait()
        ws = pltpu.async_copy(srows_v, src_out.at[pl.ds(base, _BPW)], sem_ws)
        cp_d.wait()
        wd = pltpu.async_copy(drows_v, dst_out.at[pl.ds(base, _BPW)], sem_wd)
        ws.wait()
        wd.wait()

    return gather2


_sc_gather2 = _make_sc_gather2()

_BLK = 256
_QTR = BATCH // 4


def _tc_body(d0, d1, d2, d3, src_ref, dst_ref, w_ref, out_ref):
    agg = (jnp.dot(d0[...], src_ref[:_QTR, :],
                   preferred_element_type=jnp.float32)
           + jnp.dot(d1[...], src_ref[_QTR:2 * _QTR, :],
                     preferred_element_type=jnp.float32)
           + jnp.dot(d2[...], src_ref[2 * _QTR:3 * _QTR, :],
                     preferred_element_type=jnp.float32)
           + jnp.dot(d3[...], src_ref[3 * _QTR:, :],
                     preferred_element_type=jnp.float32))
    x = (jnp.dot(agg, w_ref[:SRC_DIM, :], preferred_element_type=jnp.float32)
         + jnp.dot(dst_ref[...], w_ref[SRC_DIM:, :],
                   preferred_element_type=jnp.float32))
    out_ref[...] = jnp.maximum(x, 0.0)


def kernel(dstsrc_features, dstsrc2src, dstsrc2dst, dif_mat, w):
    src_f, dst_f = _sc_gather2(dstsrc_features, dstsrc2src, dstsrc2dst)
    out = pl.pallas_call(
        _tc_body,
        grid=(BATCH // _BLK,),
        in_specs=[
            pl.BlockSpec((_BLK, _QTR), lambda i: (i, 0)),
            pl.BlockSpec((_BLK, _QTR), lambda i: (i, 1)),
            pl.BlockSpec((_BLK, _QTR), lambda i: (i, 2)),
            pl.BlockSpec((_BLK, _QTR), lambda i: (i, 3)),
            pl.BlockSpec((BATCH, SRC_DIM), lambda i: (0, 0)),
            pl.BlockSpec((_BLK, SRC_DIM), lambda i: (i, 0)),
            pl.BlockSpec((2 * SRC_DIM, DST_DIM), lambda i: (0, 0)),
        ],
        out_specs=pl.BlockSpec((_BLK, DST_DIM), lambda i: (i, 0)),
        out_shape=jax.ShapeDtypeStruct((BATCH, DST_DIM), jnp.float32),
    )(dif_mat, dif_mat, dif_mat, dif_mat, src_f, dst_f, w)
    return out


# speedup vs baseline: 1.1392x; 1.0011x over previous
import functools

import jax
import jax.numpy as jnp
from jax import lax
from jax.experimental import pallas as pl
from jax.experimental.pallas import tpu as pltpu
from jax.experimental.pallas import tpu_sc as plsc

N_NODES = 100000
BATCH = 8192
SRC_DIM = 128
DST_DIM = 128

_NC = 2
_NS = 16
_NW = _NC * _NS
_BPW = BATCH // _NW


def _make_sc_gather2():
    mesh = plsc.VectorSubcoreMesh(core_axis_name="c", subcore_axis_name="s")

    @functools.partial(
        pl.kernel,
        mesh=mesh,
        out_type=[
            jax.ShapeDtypeStruct((BATCH, SRC_DIM), jnp.float32),
            jax.ShapeDtypeStruct((BATCH, SRC_DIM), jnp.float32),
        ],
        scratch_types=[
            pltpu.VMEM((_BPW,), jnp.int32),
            pltpu.VMEM((_BPW,), jnp.int32),
            pltpu.VMEM((_BPW, SRC_DIM), jnp.float32),
            pltpu.VMEM((_BPW, SRC_DIM), jnp.float32),
            pltpu.SemaphoreType.DMA,
            pltpu.SemaphoreType.DMA,
            pltpu.SemaphoreType.DMA,
            pltpu.SemaphoreType.DMA,
            pltpu.SemaphoreType.DMA,
            pltpu.SemaphoreType.DMA,
        ],
    )
    def gather2(table_hbm, src_idx_hbm, dst_idx_hbm, src_out, dst_out,
                sidx_v, didx_v, srows_v, drows_v,
                sem_i1, sem_i2, sem_s, sem_d, sem_ws, sem_wd):
        wid = lax.axis_index("s") * _NC + lax.axis_index("c")
        base = wid * _BPW
        ci = pltpu.async_copy(src_idx_hbm.at[pl.ds(base, _BPW)], sidx_v, sem_i1)
        cj = pltpu.async_copy(dst_idx_hbm.at[pl.ds(base, _BPW)], didx_v, sem_i2)
        ci.wait()
        cp_s = pltpu.async_copy(table_hbm.at[sidx_v], srows_v, sem_s)
        cj.wait()
        cp_d = pltpu.async_copy(table_hbm.at[didx_v], drows_v, sem_d)
        cp_s.wait()
        ws = pltpu.async_copy(srows_v, src_out.at[pl.ds(base, _BPW)], sem_ws)
        cp_d.wait()
        wd = pltpu.async_copy(drows_v, dst_out.at[pl.ds(base, _BPW)], sem_wd)
        ws.wait()
        wd.wait()

    return gather2


_sc_gather2 = _make_sc_gather2()

_BLK = 256
_QTR = BATCH // 4


def _tc_body(d0, d1, d2, d3, src_ref, dst_ref, w_ref, out_ref):
    agg = (jnp.dot(d0[...], src_ref[:_QTR, :],
                   preferred_element_type=jnp.float32)
           + jnp.dot(d1[...], src_ref[_QTR:2 * _QTR, :],
                     preferred_element_type=jnp.float32)
           + jnp.dot(d2[...], src_ref[2 * _QTR:3 * _QTR, :],
                     preferred_element_type=jnp.float32)
           + jnp.dot(d3[...], src_ref[3 * _QTR:, :],
                     preferred_element_type=jnp.float32))
    x = (jnp.dot(agg, w_ref[:SRC_DIM, :], preferred_element_type=jnp.float32)
         + jnp.dot(dst_ref[...], w_ref[SRC_DIM:, :],
                   preferred_element_type=jnp.float32))
    out_ref[...] = jnp.maximum(x, 0.0)


def kernel(dstsrc_features, dstsrc2src, dstsrc2dst, dif_mat, w):
    src_f, dst_f = _sc_gather2(dstsrc_features, dstsrc2src, dstsrc2dst)
    out = pl.pallas_call(
        _tc_body,
        grid=(BATCH // _BLK,),
        in_specs=[
            pl.BlockSpec((_BLK, _QTR), lambda i: (i, 0)),
            pl.BlockSpec((_BLK, _QTR), lambda i: (i, 1)),
            pl.BlockSpec((_BLK, _QTR), lambda i: (i, 2)),
            pl.BlockSpec((_BLK, _QTR), lambda i: (i, 3)),
            pl.BlockSpec((BATCH, SRC_DIM), lambda i: (0, 0)),
            pl.BlockSpec((_BLK, SRC_DIM), lambda i: (i, 0)),
            pl.BlockSpec((2 * SRC_DIM, DST_DIM), lambda i: (0, 0)),
        ],
        out_specs=pl.BlockSpec((_BLK, DST_DIM), lambda i: (i, 0)),
        out_shape=jax.ShapeDtypeStruct((BATCH, DST_DIM), jnp.float32),
    )(dif_mat, dif_mat, dif_mat, dif_mat, src_f, dst_f, w)
    return out
